# baseline (speedup 1.0000x reference)
import functools

import jax
import jax.numpy as jnp
from jax.experimental import pallas as pl
from jax.experimental.pallas import tpu as pltpu
from jax.experimental.pallas import tpu_sc as plsc

B, N, M = 4, 4096, 1024
C = 256
IN_C = 2 * C
EPS_BN = 1e-5

TN_NN = 256
TN_MM = 512
NW = 32
GW = 128


def _nn_body(src_ref, dstT_ref, idx_ref, w_ref):
    b = pl.program_id(0)
    s = src_ref[0]
    t = dstT_ref[0]
    dx = s[:, 0:1] - t[0:1, :]
    dy = s[:, 1:2] - t[1:2, :]
    dz = s[:, 2:3] - t[2:3, :]
    d2 = dx * dx + dy * dy + dz * dz
    inf = jnp.float32(jnp.inf)
    v1 = jnp.min(d2, axis=1, keepdims=True)
    v2 = jnp.min(jnp.where(d2 > v1, d2, inf), axis=1, keepdims=True)
    v3 = jnp.min(jnp.where(d2 > v2, d2, inf), axis=1, keepdims=True)
    iota = jax.lax.broadcasted_iota(jnp.int32, d2.shape, 1).astype(jnp.float32)
    i1 = jnp.sum(jnp.where(d2 == v1, iota, 0.0), axis=1, keepdims=True)
    i2 = jnp.sum(jnp.where(d2 == v2, iota, 0.0), axis=1, keepdims=True)
    i3 = jnp.sum(jnp.where(d2 == v3, iota, 0.0), axis=1, keepdims=True)
    ii = jnp.concatenate([i1, i2, i3], axis=1).astype(jnp.int32)
    ii = jnp.minimum(ii, M - 1)
    vv = jnp.concatenate([v1, v2, v3], axis=1)
    d3 = jnp.sqrt(vv) + 1e-8
    w = 1.0 / d3
    w = w / jnp.sum(w, axis=1, keepdims=True)
    idx_ref[0] = ii.T + b * M
    w_ref[0] = w


def _three_nn(xyz_src, xyz_dstT):
    return pl.pallas_call(
        _nn_body,
        grid=(B, N // TN_NN),
        in_specs=[
            pl.BlockSpec((1, TN_NN, 3), lambda b, i: (b, i, 0)),
            pl.BlockSpec((1, 3, M), lambda b, i: (b, 0, 0)),
        ],
        out_specs=[
            pl.BlockSpec((1, 3, TN_NN), lambda b, i: (b, 0, i)),
            pl.BlockSpec((1, TN_NN, 3), lambda b, i: (b, i, 0)),
        ],
        out_shape=[
            jax.ShapeDtypeStruct((B, 3, N), jnp.int32),
            jax.ShapeDtypeStruct((B, N, 3), jnp.float32),
        ],
    )(xyz_src, xyz_dstT)


def _sc_gather(table, gidx):
    NI = gidx.shape[0]
    per_w = NI // NW
    nch = per_w // GW
    mesh = plsc.VectorSubcoreMesh(core_axis_name="c", subcore_axis_name="s")

    @functools.partial(
        pl.kernel,
        mesh=mesh,
        out_type=jax.ShapeDtypeStruct((NI, C // 2), jnp.int32),
        scratch_types=[
            pltpu.VMEM((per_w,), jnp.int32),
            pltpu.VMEM((GW, C // 2), jnp.int32),
            pltpu.VMEM((GW, C // 2), jnp.int32),
            pltpu.SemaphoreType.DMA,
            pltpu.SemaphoreType.DMA,
            pltpu.SemaphoreType.DMA,
            pltpu.SemaphoreType.DMA,
        ],
    )
    def k(table_hbm, idx_hbm, out_hbm, idx_v, rows0, rows1,
          gsem0, gsem1, wsem0, wsem1):
        wid = jax.lax.axis_index("s") * 2 + jax.lax.axis_index("c")
        base = wid * per_w
        pltpu.sync_copy(idx_hbm.at[pl.ds(base, per_w)], idx_v)
        rows = (rows0, rows1)
        gsem = (gsem0, gsem1)
        wsem = (wsem0, wsem1)
        gcp = [None, None]
        wcp = [None, None]
        for c in range(nch):
            s = c % 2
            if c >= 2:
                wcp[s].wait()
            gcp[s] = pltpu.async_copy(
                table_hbm.at[idx_v.at[pl.ds(c * GW, GW)]], rows[s], gsem[s])
            if c >= 1:
                t = (c - 1) % 2
                gcp[t].wait()
                wcp[t] = pltpu.async_copy(
                    rows[t], out_hbm.at[pl.ds(base + (c - 1) * GW, GW)],
                    wsem[t])
        s = (nch - 1) % 2
        gcp[s].wait()
        pltpu.sync_copy(rows[s], out_hbm.at[pl.ds(base + (nch - 1) * GW, GW)])
        wcp[(nch - 2) % 2].wait()

    return k(table, gidx)


def _l1a_body(fsrc_ref, wbt_ref, p_ref):
    p_ref[...] = jnp.dot(fsrc_ref[...].astype(jnp.bfloat16), wbt_ref[...],
                         preferred_element_type=jnp.float32)


def _layer1a(fsrcT, W0bT16):
    steps = (B * N) // TN_MM
    return pl.pallas_call(
        _l1a_body,
        grid=(steps,),
        in_specs=[
            pl.BlockSpec((TN_MM, C), lambda i: (i, 0)),
            pl.BlockSpec((C, C), lambda i: (0, 0)),
        ],
        out_specs=pl.BlockSpec((TN_MM, C), lambda i: (i, 0)),
        out_shape=jax.ShapeDtypeStruct((B * N, C), jnp.float32),
    )(fsrcT, W0bT16)


def _l1b_body(g_ref, w_ref, part_ref, wat_ref, b0_ref, y_ref, ps_ref, pss_ref):
    w = w_ref[0]
    g = g_ref[0].astype(jnp.float32)
    interp = (g[0] * w[:, 0:1] + g[1] * w[:, 1:2]
              + g[2] * w[:, 2:3])
    y = jnp.dot(interp.astype(jnp.bfloat16), wat_ref[...],
                preferred_element_type=jnp.float32)
    y = y + part_ref[0] + b0_ref[...]
    y_ref[0] = y

    @pl.when((pl.program_id(0) == 0) & (pl.program_id(1) == 0))
    def _():
        ps_ref[...] = jnp.zeros_like(ps_ref)
        pss_ref[...] = jnp.zeros_like(pss_ref)

    ps_ref[...] += jnp.sum(y, axis=0, keepdims=True)
    pss_ref[...] += jnp.sum(y * y, axis=0, keepdims=True)


def _layer1b(gathered, w, part, W0aT16, b0row):
    steps = N // TN_MM
    return pl.pallas_call(
        _l1b_body,
        grid=(B, steps),
        in_specs=[
            pl.BlockSpec((1, 3, TN_MM, C), lambda b, i: (b, 0, i, 0)),
            pl.BlockSpec((1, TN_MM, 3), lambda b, i: (b, i, 0)),
            pl.BlockSpec((1, TN_MM, C), lambda b, i: (b, i, 0)),
            pl.BlockSpec((C, C), lambda b, i: (0, 0)),
            pl.BlockSpec((1, C), lambda b, i: (0, 0)),
        ],
        out_specs=[
            pl.BlockSpec((1, TN_MM, C), lambda b, i: (b, i, 0)),
            pl.BlockSpec((1, C), lambda b, i: (0, 0)),
            pl.BlockSpec((1, C), lambda b, i: (0, 0)),
        ],
        out_shape=[
            jax.ShapeDtypeStruct((B, N, C), jnp.float32),
            jax.ShapeDtypeStruct((1, C), jnp.float32),
            jax.ShapeDtypeStruct((1, C), jnp.float32),
        ],
    )(gathered, w, part, W0aT16, b0row)


def _l2_body(y0_ref, sc_ref, sh_ref, w1t_ref, b1_ref, y_ref, ps_ref, pss_ref):
    h = jnp.maximum(y0_ref[0] * sc_ref[...] + sh_ref[...], 0.0)
    y = jnp.dot(h.astype(jnp.bfloat16), w1t_ref[...],
                preferred_element_type=jnp.float32)
    y = y + b1_ref[...]
    y_ref[0] = y.T

    @pl.when((pl.program_id(0) == 0) & (pl.program_id(1) == 0))
    def _():
        ps_ref[...] = jnp.zeros_like(ps_ref)
        pss_ref[...] = jnp.zeros_like(pss_ref)

    ps_ref[...] += jnp.sum(y, axis=0, keepdims=True)
    pss_ref[...] += jnp.sum(y * y, axis=0, keepdims=True)


def _layer2(y0, sc0, sh0, W1T16, b1row):
    steps = N // TN_MM
    return pl.pallas_call(
        _l2_body,
        grid=(B, steps),
        in_specs=[
            pl.BlockSpec((1, TN_MM, C), lambda b, i: (b, i, 0)),
            pl.BlockSpec((1, C), lambda b, i: (0, 0)),
            pl.BlockSpec((1, C), lambda b, i: (0, 0)),
            pl.BlockSpec((C, C), lambda b, i: (0, 0)),
            pl.BlockSpec((1, C), lambda b, i: (0, 0)),
        ],
        out_specs=[
            pl.BlockSpec((1, C, TN_MM), lambda b, i: (b, 0, i)),
            pl.BlockSpec((1, C), lambda b, i: (0, 0)),
            pl.BlockSpec((1, C), lambda b, i: (0, 0)),
        ],
        out_shape=[
            jax.ShapeDtypeStruct((B, C, N), jnp.float32),
            jax.ShapeDtypeStruct((1, C), jnp.float32),
            jax.ShapeDtypeStruct((1, C), jnp.float32),
        ],
    )(y0, sc0, sh0, W1T16, b1row)


def _out_body(y1_ref, sc_ref, sh_ref, o_ref):
    o_ref[0] = jnp.maximum(y1_ref[0] * sc_ref[...] + sh_ref[...], 0.0)


def _finalize(y1, sc1, sh1):
    TF = 2048
    return pl.pallas_call(
        _out_body,
        grid=(B, N // TF),
        in_specs=[
            pl.BlockSpec((1, C, TF), lambda b, i: (b, 0, i)),
            pl.BlockSpec((C, 1), lambda b, i: (0, 0)),
            pl.BlockSpec((C, 1), lambda b, i: (0, 0)),
        ],
        out_specs=pl.BlockSpec((1, C, TF), lambda b, i: (b, 0, i)),
        out_shape=jax.ShapeDtypeStruct((B, C, N), jnp.float32),
    )(y1, sc1, sh1)


def kernel(xyz_src, xyz_dst, feat_src, feat_dst,
           W0, b0, gamma0, beta0, W1, b1, gamma1, beta1):
    xyz_dstT = jnp.transpose(xyz_dst, (0, 2, 1))
    table32 = jax.lax.bitcast_convert_type(
        jnp.transpose(feat_dst, (0, 2, 1))
        .astype(jnp.bfloat16).reshape(B * M, C // 2, 2),
        jnp.int32)
    fsrcT = jnp.transpose(feat_src, (0, 2, 1)).reshape(B * N, C)
    W0T = W0.T
    W0aT16 = W0T[:C].astype(jnp.bfloat16)
    W0bT16 = W0T[C:].astype(jnp.bfloat16)

    idxT, w = _three_nn(xyz_src, xyz_dstT)
    g32 = _sc_gather(table32, idxT.reshape(B * 3 * N))
    gathered = (jax.lax.bitcast_convert_type(g32, jnp.bfloat16)
                .reshape(B, 3, N, C))

    part = _layer1a(fsrcT, W0bT16)
    y0, ps0, pss0 = _layer1b(gathered, w, part.reshape(B, N, C),
                             W0aT16, b0.reshape(1, C))

    n = jnp.float32(B * N)
    mu0 = ps0 / n
    var0 = pss0 / n - mu0 * mu0
    sc0 = gamma0.reshape(1, C) / jnp.sqrt(var0 + EPS_BN)
    sh0 = beta0.reshape(1, C) - mu0 * sc0

    y1, ps1, pss1 = _layer2(y0, sc0, sh0,
                            W1.T.astype(jnp.bfloat16), b1.reshape(1, C))
    mu1 = ps1 / n
    var1 = pss1 / n - mu1 * mu1
    sc1 = gamma1.reshape(1, C) / jnp.sqrt(var1 + EPS_BN)
    sh1 = beta1.reshape(1, C) - mu1 * sc1

    return _finalize(y1, sc1.reshape(C, 1), sh1.reshape(C, 1))


# speedup vs baseline: 1.9317x; 1.9317x over previous
import functools

import jax
import jax.numpy as jnp
from jax.experimental import pallas as pl
from jax.experimental.pallas import tpu as pltpu
from jax.experimental.pallas import tpu_sc as plsc

B, N, M = 4, 4096, 1024
C = 256
IN_C = 2 * C
EPS_BN = 1e-5

TN_NN = 256
TN_MM = 512
NW = 32
GW = 128


def _nn_body(src_ref, dstT_ref, idx_ref, w_ref):
    b = pl.program_id(0)
    s = src_ref[0]
    t = dstT_ref[0]
    dx = s[:, 0:1] - t[0:1, :]
    dy = s[:, 1:2] - t[1:2, :]
    dz = s[:, 2:3] - t[2:3, :]
    d2 = dx * dx + dy * dy + dz * dz
    inf = jnp.float32(jnp.inf)
    v1 = jnp.min(d2, axis=1, keepdims=True)
    v2 = jnp.min(jnp.where(d2 > v1, d2, inf), axis=1, keepdims=True)
    v3 = jnp.min(jnp.where(d2 > v2, d2, inf), axis=1, keepdims=True)
    iota = jax.lax.broadcasted_iota(jnp.int32, d2.shape, 1).astype(jnp.float32)
    i1 = jnp.sum(jnp.where(d2 == v1, iota, 0.0), axis=1, keepdims=True)
    i2 = jnp.sum(jnp.where(d2 == v2, iota, 0.0), axis=1, keepdims=True)
    i3 = jnp.sum(jnp.where(d2 == v3, iota, 0.0), axis=1, keepdims=True)
    ii = jnp.concatenate([i1, i2, i3], axis=1).astype(jnp.int32)
    ii = jnp.minimum(ii, M - 1)
    vv = jnp.concatenate([v1, v2, v3], axis=1)
    d3 = jnp.sqrt(vv) + 1e-8
    w = 1.0 / d3
    w = w / jnp.sum(w, axis=1, keepdims=True)
    idx_ref[0] = ii.T + b * M
    w_ref[0] = w


def _three_nn(xyz_src, xyz_dstT):
    return pl.pallas_call(
        _nn_body,
        grid=(B, N // TN_NN),
        in_specs=[
            pl.BlockSpec((1, TN_NN, 3), lambda b, i: (b, i, 0)),
            pl.BlockSpec((1, 3, M), lambda b, i: (b, 0, 0)),
        ],
        out_specs=[
            pl.BlockSpec((1, 3, TN_NN), lambda b, i: (b, 0, i)),
            pl.BlockSpec((1, TN_NN, 3), lambda b, i: (b, i, 0)),
        ],
        out_shape=[
            jax.ShapeDtypeStruct((B, 3, N), jnp.int32),
            jax.ShapeDtypeStruct((B, N, 3), jnp.float32),
        ],
    )(xyz_src, xyz_dstT)


def _sc_gather(table, gidx):
    NI = gidx.shape[0]
    per_w = NI // NW
    nch = per_w // GW
    mesh = plsc.VectorSubcoreMesh(core_axis_name="c", subcore_axis_name="s")

    @functools.partial(
        pl.kernel,
        mesh=mesh,
        out_type=jax.ShapeDtypeStruct((NI, C // 2), jnp.int32),
        scratch_types=[
            pltpu.VMEM((per_w,), jnp.int32),
            pltpu.VMEM((GW, C // 2), jnp.int32),
            pltpu.VMEM((GW, C // 2), jnp.int32),
            pltpu.SemaphoreType.DMA,
            pltpu.SemaphoreType.DMA,
            pltpu.SemaphoreType.DMA,
            pltpu.SemaphoreType.DMA,
        ],
    )
    def k(table_hbm, idx_hbm, out_hbm, idx_v, rows0, rows1,
          gsem0, gsem1, wsem0, wsem1):
        wid = jax.lax.axis_index("s") * 2 + jax.lax.axis_index("c")
        base = wid * per_w
        pltpu.sync_copy(idx_hbm.at[pl.ds(base, per_w)], idx_v)
        rows = (rows0, rows1)
        gsem = (gsem0, gsem1)
        wsem = (wsem0, wsem1)
        gcp = [None, None]
        wcp = [None, None]
        for c in range(nch):
            s = c % 2
            if c >= 2:
                wcp[s].wait()
            gcp[s] = pltpu.async_copy(
                table_hbm.at[idx_v.at[pl.ds(c * GW, GW)]], rows[s], gsem[s])
            if c >= 1:
                t = (c - 1) % 2
                gcp[t].wait()
                wcp[t] = pltpu.async_copy(
                    rows[t], out_hbm.at[pl.ds(base + (c - 1) * GW, GW)],
                    wsem[t])
        s = (nch - 1) % 2
        gcp[s].wait()
        pltpu.sync_copy(rows[s], out_hbm.at[pl.ds(base + (nch - 1) * GW, GW)])
        wcp[(nch - 2) % 2].wait()

    return k(table, gidx)


def _l1a_body(fsrc_ref, wbt_ref, p_ref):
    p_ref[...] = jnp.dot(fsrc_ref[...].astype(jnp.bfloat16), wbt_ref[...],
                         preferred_element_type=jnp.float32)


def _layer1a(fsrcT, W0bT16):
    steps = (B * N) // TN_MM
    return pl.pallas_call(
        _l1a_body,
        grid=(steps,),
        in_specs=[
            pl.BlockSpec((TN_MM, C), lambda i: (i, 0)),
            pl.BlockSpec((C, C), lambda i: (0, 0)),
        ],
        out_specs=pl.BlockSpec((TN_MM, C), lambda i: (i, 0)),
        out_shape=jax.ShapeDtypeStruct((B * N, C), jnp.float32),
    )(fsrcT, W0bT16)


def _l1b_body(g_ref, w_ref, part_ref, wae_ref, wao_ref, b0_ref,
              y_ref, ps_ref, pss_ref):
    w = w_ref[0]
    g = g_ref[0]
    ge = jax.lax.bitcast_convert_type(g << 16, jnp.float32)
    go = jax.lax.bitcast_convert_type(g & jnp.int32(-65536), jnp.float32)
    ie = (ge[0] * w[:, 0:1] + ge[1] * w[:, 1:2] + ge[2] * w[:, 2:3])
    io = (go[0] * w[:, 0:1] + go[1] * w[:, 1:2] + go[2] * w[:, 2:3])
    y = jnp.dot(ie.astype(jnp.bfloat16), wae_ref[...],
                preferred_element_type=jnp.float32)
    y = y + jnp.dot(io.astype(jnp.bfloat16), wao_ref[...],
                    preferred_element_type=jnp.float32)
    y = y + part_ref[0] + b0_ref[...]
    y_ref[0] = y

    @pl.when((pl.program_id(0) == 0) & (pl.program_id(1) == 0))
    def _():
        ps_ref[...] = jnp.zeros_like(ps_ref)
        pss_ref[...] = jnp.zeros_like(pss_ref)

    ps_ref[...] += jnp.sum(y, axis=0, keepdims=True)
    pss_ref[...] += jnp.sum(y * y, axis=0, keepdims=True)


def _layer1b(gathered, w, part, WaE16, WaO16, b0row):
    steps = N // TN_MM
    return pl.pallas_call(
        _l1b_body,
        grid=(B, steps),
        in_specs=[
            pl.BlockSpec((1, 3, TN_MM, C // 2), lambda b, i: (b, 0, i, 0)),
            pl.BlockSpec((1, TN_MM, 3), lambda b, i: (b, i, 0)),
            pl.BlockSpec((1, TN_MM, C), lambda b, i: (b, i, 0)),
            pl.BlockSpec((C // 2, C), lambda b, i: (0, 0)),
            pl.BlockSpec((C // 2, C), lambda b, i: (0, 0)),
            pl.BlockSpec((1, C), lambda b, i: (0, 0)),
        ],
        out_specs=[
            pl.BlockSpec((1, TN_MM, C), lambda b, i: (b, i, 0)),
            pl.BlockSpec((1, C), lambda b, i: (0, 0)),
            pl.BlockSpec((1, C), lambda b, i: (0, 0)),
        ],
        out_shape=[
            jax.ShapeDtypeStruct((B, N, C), jnp.float32),
            jax.ShapeDtypeStruct((1, C), jnp.float32),
            jax.ShapeDtypeStruct((1, C), jnp.float32),
        ],
    )(gathered, w, part, WaE16, WaO16, b0row)


def _l2_body(y0_ref, sc_ref, sh_ref, w1t_ref, b1_ref, y_ref, ps_ref, pss_ref):
    h = jnp.maximum(y0_ref[0] * sc_ref[...] + sh_ref[...], 0.0)
    y = jnp.dot(h.astype(jnp.bfloat16), w1t_ref[...],
                preferred_element_type=jnp.float32)
    y = y + b1_ref[...]
    y_ref[0] = y.T

    @pl.when((pl.program_id(0) == 0) & (pl.program_id(1) == 0))
    def _():
        ps_ref[...] = jnp.zeros_like(ps_ref)
        pss_ref[...] = jnp.zeros_like(pss_ref)

    ps_ref[...] += jnp.sum(y, axis=0, keepdims=True)
    pss_ref[...] += jnp.sum(y * y, axis=0, keepdims=True)


def _layer2(y0, sc0, sh0, W1T16, b1row):
    steps = N // TN_MM
    return pl.pallas_call(
        _l2_body,
        grid=(B, steps),
        in_specs=[
            pl.BlockSpec((1, TN_MM, C), lambda b, i: (b, i, 0)),
            pl.BlockSpec((1, C), lambda b, i: (0, 0)),
            pl.BlockSpec((1, C), lambda b, i: (0, 0)),
            pl.BlockSpec((C, C), lambda b, i: (0, 0)),
            pl.BlockSpec((1, C), lambda b, i: (0, 0)),
        ],
        out_specs=[
            pl.BlockSpec((1, C, TN_MM), lambda b, i: (b, 0, i)),
            pl.BlockSpec((1, C), lambda b, i: (0, 0)),
            pl.BlockSpec((1, C), lambda b, i: (0, 0)),
        ],
        out_shape=[
            jax.ShapeDtypeStruct((B, C, N), jnp.float32),
            jax.ShapeDtypeStruct((1, C), jnp.float32),
            jax.ShapeDtypeStruct((1, C), jnp.float32),
        ],
    )(y0, sc0, sh0, W1T16, b1row)


def _out_body(y1_ref, sc_ref, sh_ref, o_ref):
    o_ref[0] = jnp.maximum(y1_ref[0] * sc_ref[...] + sh_ref[...], 0.0)


def _finalize(y1, sc1, sh1):
    TF = 2048
    return pl.pallas_call(
        _out_body,
        grid=(B, N // TF),
        in_specs=[
            pl.BlockSpec((1, C, TF), lambda b, i: (b, 0, i)),
            pl.BlockSpec((C, 1), lambda b, i: (0, 0)),
            pl.BlockSpec((C, 1), lambda b, i: (0, 0)),
        ],
        out_specs=pl.BlockSpec((1, C, TF), lambda b, i: (b, 0, i)),
        out_shape=jax.ShapeDtypeStruct((B, C, N), jnp.float32),
    )(y1, sc1, sh1)


def kernel(xyz_src, xyz_dst, feat_src, feat_dst,
           W0, b0, gamma0, beta0, W1, b1, gamma1, beta1):
    xyz_dstT = jnp.transpose(xyz_dst, (0, 2, 1))
    table32 = jax.lax.bitcast_convert_type(
        jnp.transpose(feat_dst, (0, 2, 1))
        .astype(jnp.bfloat16).reshape(B * M, C // 2, 2),
        jnp.int32)
    fsrcT = jnp.transpose(feat_src, (0, 2, 1)).reshape(B * N, C)
    W0T = W0.T
    WaE16 = W0T[:C:2].astype(jnp.bfloat16)
    WaO16 = W0T[1:C:2].astype(jnp.bfloat16)
    W0bT16 = W0T[C:].astype(jnp.bfloat16)

    idxT, w = _three_nn(xyz_src, xyz_dstT)
    g32 = _sc_gather(table32, idxT.reshape(B * 3 * N))
    gathered = g32.reshape(B, 3, N, C // 2)

    part = _layer1a(fsrcT, W0bT16)
    y0, ps0, pss0 = _layer1b(gathered, w, part.reshape(B, N, C),
                             WaE16, WaO16, b0.reshape(1, C))

    n = jnp.float32(B * N)
    mu0 = ps0 / n
    var0 = pss0 / n - mu0 * mu0
    sc0 = gamma0.reshape(1, C) / jnp.sqrt(var0 + EPS_BN)
    sh0 = beta0.reshape(1, C) - mu0 * sc0

    y1, ps1, pss1 = _layer2(y0, sc0, sh0,
                            W1.T.astype(jnp.bfloat16), b1.reshape(1, C))
    mu1 = ps1 / n
    var1 = pss1 / n - mu1 * mu1
    sc1 = gamma1.reshape(1, C) / jnp.sqrt(var1 + EPS_BN)
    sh1 = beta1.reshape(1, C) - mu1 * sc1

    return _finalize(y1, sc1.reshape(C, 1), sh1.reshape(C, 1))


# speedup vs baseline: 2.1498x; 1.1129x over previous
import functools

import jax
import jax.numpy as jnp
from jax.experimental import pallas as pl
from jax.experimental.pallas import tpu as pltpu
from jax.experimental.pallas import tpu_sc as plsc

B, N, M = 4, 4096, 1024
C = 256
IN_C = 2 * C
EPS_BN = 1e-5

TN_NN = 512
TN_MM = 512
NW = 32
GW = 128


def _nn_body(src_ref, dstT_ref, fd_ref, idx_ref, w_ref, tab_ref):
    b = pl.program_id(0)
    s = src_ref[0]
    t = dstT_ref[0]

    x = fd_ref[0].T
    v = jax.lax.bitcast_convert_type(x, jnp.int32)
    r = ((v >> 16) & 1) + jnp.int32(0x7FFF)
    b16 = ((v + r) >> 16) & jnp.int32(0xFFFF)
    half = C // 2
    tab_ref[...] = b16[:, :half] | (b16[:, half:] << 16)
    dx = s[:, 0:1] - t[0:1, :]
    dy = s[:, 1:2] - t[1:2, :]
    dz = s[:, 2:3] - t[2:3, :]
    d2 = dx * dx + dy * dy + dz * dz
    inf = jnp.float32(jnp.inf)
    v1 = jnp.min(d2, axis=1, keepdims=True)
    v2 = jnp.min(jnp.where(d2 > v1, d2, inf), axis=1, keepdims=True)
    v3 = jnp.min(jnp.where(d2 > v2, d2, inf), axis=1, keepdims=True)
    iota = jax.lax.broadcasted_iota(jnp.int32, d2.shape, 1).astype(jnp.float32)
    i1 = jnp.sum(jnp.where(d2 == v1, iota, 0.0), axis=1, keepdims=True)
    i2 = jnp.sum(jnp.where(d2 == v2, iota, 0.0), axis=1, keepdims=True)
    i3 = jnp.sum(jnp.where(d2 == v3, iota, 0.0), axis=1, keepdims=True)
    ii = jnp.concatenate([i1, i2, i3], axis=1).astype(jnp.int32)
    ii = jnp.minimum(ii, M - 1)
    vv = jnp.concatenate([v1, v2, v3], axis=1)
    d3 = jnp.sqrt(vv) + 1e-8
    w = 1.0 / d3
    w = w / jnp.sum(w, axis=1, keepdims=True)
    idx_ref[0] = ii.T + b * M
    w_ref[0] = w


def _three_nn(xyz_src, xyz_dstT, feat_dst):
    steps = N // TN_NN
    TM = M // steps
    return pl.pallas_call(
        _nn_body,
        grid=(B, steps),
        in_specs=[
            pl.BlockSpec((1, TN_NN, 3), lambda b, i: (b, i, 0)),
            pl.BlockSpec((1, 3, M), lambda b, i: (b, 0, 0)),
            pl.BlockSpec((1, C, TM), lambda b, i: (b, 0, i)),
        ],
        out_specs=[
            pl.BlockSpec((1, 3, TN_NN), lambda b, i: (b, 0, i)),
            pl.BlockSpec((1, TN_NN, 3), lambda b, i: (b, i, 0)),
            pl.BlockSpec((TM, C // 2), lambda b, i, s=steps: (b * s + i, 0)),
        ],
        out_shape=[
            jax.ShapeDtypeStruct((B, 3, N), jnp.int32),
            jax.ShapeDtypeStruct((B, N, 3), jnp.float32),
            jax.ShapeDtypeStruct((B * M, C // 2), jnp.int32),
        ],
    )(xyz_src, xyz_dstT, feat_dst)


def _sc_gather(table, gidx):
    NI = gidx.shape[0]
    per_w = NI // NW
    nch = per_w // GW
    mesh = plsc.VectorSubcoreMesh(core_axis_name="c", subcore_axis_name="s")

    @functools.partial(
        pl.kernel,
        mesh=mesh,
        out_type=jax.ShapeDtypeStruct((NI, C // 2), jnp.int32),
        scratch_types=[
            pltpu.VMEM((per_w,), jnp.int32),
            pltpu.VMEM((GW, C // 2), jnp.int32),
            pltpu.VMEM((GW, C // 2), jnp.int32),
            pltpu.SemaphoreType.DMA,
            pltpu.SemaphoreType.DMA,
            pltpu.SemaphoreType.DMA,
            pltpu.SemaphoreType.DMA,
        ],
    )
    def k(table_hbm, idx_hbm, out_hbm, idx_v, rows0, rows1,
          gsem0, gsem1, wsem0, wsem1):
        wid = jax.lax.axis_index("s") * 2 + jax.lax.axis_index("c")
        base = wid * per_w
        pltpu.sync_copy(idx_hbm.at[pl.ds(base, per_w)], idx_v)
        rows = (rows0, rows1)
        gsem = (gsem0, gsem1)
        wsem = (wsem0, wsem1)
        gcp = [None, None]
        wcp = [None, None]
        for c in range(nch):
            s = c % 2
            if c >= 2:
                wcp[s].wait()
            gcp[s] = pltpu.async_copy(
                table_hbm.at[idx_v.at[pl.ds(c * GW, GW)]], rows[s], gsem[s])
            if c >= 1:
                t = (c - 1) % 2
                gcp[t].wait()
                wcp[t] = pltpu.async_copy(
                    rows[t], out_hbm.at[pl.ds(base + (c - 1) * GW, GW)],
                    wsem[t])
        s = (nch - 1) % 2
        gcp[s].wait()
        pltpu.sync_copy(rows[s], out_hbm.at[pl.ds(base + (nch - 1) * GW, GW)])
        wcp[(nch - 2) % 2].wait()

    return k(table, gidx)


def _l1a_body(fsrc_ref, wbt_ref, p_ref):
    p_ref[...] = jnp.dot(fsrc_ref[...].astype(jnp.bfloat16), wbt_ref[...],
                         preferred_element_type=jnp.float32)


def _layer1a(fsrcT, W0bT16):
    steps = (B * N) // TN_MM
    return pl.pallas_call(
        _l1a_body,
        grid=(steps,),
        in_specs=[
            pl.BlockSpec((TN_MM, C), lambda i: (i, 0)),
            pl.BlockSpec((C, C), lambda i: (0, 0)),
        ],
        out_specs=pl.BlockSpec((TN_MM, C), lambda i: (i, 0)),
        out_shape=jax.ShapeDtypeStruct((B * N, C), jnp.float32),
    )(fsrcT, W0bT16)


def _l1b_body(g_ref, w_ref, part_ref, wae_ref, wao_ref, b0_ref,
              y_ref, ps_ref, pss_ref):
    w = w_ref[0]
    g = g_ref[0]
    ge = jax.lax.bitcast_convert_type(g << 16, jnp.float32)
    go = jax.lax.bitcast_convert_type(g & jnp.int32(-65536), jnp.float32)
    ie = (ge[0] * w[:, 0:1] + ge[1] * w[:, 1:2] + ge[2] * w[:, 2:3])
    io = (go[0] * w[:, 0:1] + go[1] * w[:, 1:2] + go[2] * w[:, 2:3])
    y = jnp.dot(ie.astype(jnp.bfloat16), wae_ref[...],
                preferred_element_type=jnp.float32)
    y = y + jnp.dot(io.astype(jnp.bfloat16), wao_ref[...],
                    preferred_element_type=jnp.float32)
    y = y + part_ref[0] + b0_ref[...]
    y_ref[0] = y

    @pl.when((pl.program_id(0) == 0) & (pl.program_id(1) == 0))
    def _():
        ps_ref[...] = jnp.zeros_like(ps_ref)
        pss_ref[...] = jnp.zeros_like(pss_ref)

    ps_ref[...] += jnp.sum(y, axis=0, keepdims=True)
    pss_ref[...] += jnp.sum(y * y, axis=0, keepdims=True)


def _layer1b(gathered, w, part, WaE16, WaO16, b0row):
    steps = N // TN_MM
    return pl.pallas_call(
        _l1b_body,
        grid=(B, steps),
        in_specs=[
            pl.BlockSpec((1, 3, TN_MM, C // 2), lambda b, i: (b, 0, i, 0)),
            pl.BlockSpec((1, TN_MM, 3), lambda b, i: (b, i, 0)),
            pl.BlockSpec((1, TN_MM, C), lambda b, i: (b, i, 0)),
            pl.BlockSpec((C // 2, C), lambda b, i: (0, 0)),
            pl.BlockSpec((C // 2, C), lambda b, i: (0, 0)),
            pl.BlockSpec((1, C), lambda b, i: (0, 0)),
        ],
        out_specs=[
            pl.BlockSpec((1, TN_MM, C), lambda b, i: (b, i, 0)),
            pl.BlockSpec((1, C), lambda b, i: (0, 0)),
            pl.BlockSpec((1, C), lambda b, i: (0, 0)),
        ],
        out_shape=[
            jax.ShapeDtypeStruct((B, N, C), jnp.float32),
            jax.ShapeDtypeStruct((1, C), jnp.float32),
            jax.ShapeDtypeStruct((1, C), jnp.float32),
        ],
    )(gathered, w, part, WaE16, WaO16, b0row)


def _l2_body(y0_ref, sc_ref, sh_ref, w1t_ref, b1_ref, y_ref, ps_ref, pss_ref):
    h = jnp.maximum(y0_ref[0] * sc_ref[...] + sh_ref[...], 0.0)
    y = jnp.dot(h.astype(jnp.bfloat16), w1t_ref[...],
                preferred_element_type=jnp.float32)
    y = y + b1_ref[...]
    y_ref[0] = y.T

    @pl.when((pl.program_id(0) == 0) & (pl.program_id(1) == 0))
    def _():
        ps_ref[...] = jnp.zeros_like(ps_ref)
        pss_ref[...] = jnp.zeros_like(pss_ref)

    ps_ref[...] += jnp.sum(y, axis=0, keepdims=True)
    pss_ref[...] += jnp.sum(y * y, axis=0, keepdims=True)


def _layer2(y0, sc0, sh0, W1T16, b1row):
    steps = N // TN_MM
    return pl.pallas_call(
        _l2_body,
        grid=(B, steps),
        in_specs=[
            pl.BlockSpec((1, TN_MM, C), lambda b, i: (b, i, 0)),
            pl.BlockSpec((1, C), lambda b, i: (0, 0)),
            pl.BlockSpec((1, C), lambda b, i: (0, 0)),
            pl.BlockSpec((C, C), lambda b, i: (0, 0)),
            pl.BlockSpec((1, C), lambda b, i: (0, 0)),
        ],
        out_specs=[
            pl.BlockSpec((1, C, TN_MM), lambda b, i: (b, 0, i)),
            pl.BlockSpec((1, C), lambda b, i: (0, 0)),
            pl.BlockSpec((1, C), lambda b, i: (0, 0)),
        ],
        out_shape=[
            jax.ShapeDtypeStruct((B, C, N), jnp.float32),
            jax.ShapeDtypeStruct((1, C), jnp.float32),
            jax.ShapeDtypeStruct((1, C), jnp.float32),
        ],
    )(y0, sc0, sh0, W1T16, b1row)


def _out_body(y1_ref, sc_ref, sh_ref, o_ref):
    o_ref[0] = jnp.maximum(y1_ref[0] * sc_ref[...] + sh_ref[...], 0.0)


def _finalize(y1, sc1, sh1):
    TF = 2048
    return pl.pallas_call(
        _out_body,
        grid=(B, N // TF),
        in_specs=[
            pl.BlockSpec((1, C, TF), lambda b, i: (b, 0, i)),
            pl.BlockSpec((C, 1), lambda b, i: (0, 0)),
            pl.BlockSpec((C, 1), lambda b, i: (0, 0)),
        ],
        out_specs=pl.BlockSpec((1, C, TF), lambda b, i: (b, 0, i)),
        out_shape=jax.ShapeDtypeStruct((B, C, N), jnp.float32),
    )(y1, sc1, sh1)


def kernel(xyz_src, xyz_dst, feat_src, feat_dst,
           W0, b0, gamma0, beta0, W1, b1, gamma1, beta1):
    xyz_dstT = jnp.transpose(xyz_dst, (0, 2, 1))
    fsrcT = jnp.transpose(feat_src, (0, 2, 1)).reshape(B * N, C)
    W0T = W0.T
    WaE16 = W0T[:C // 2].astype(jnp.bfloat16)
    WaO16 = W0T[C // 2:C].astype(jnp.bfloat16)
    W0bT16 = W0T[C:].astype(jnp.bfloat16)

    idxT, w, table32 = _three_nn(xyz_src, xyz_dstT, feat_dst)
    g32 = _sc_gather(table32, idxT.reshape(B * 3 * N))
    gathered = g32.reshape(B, 3, N, C // 2)

    part = _layer1a(fsrcT, W0bT16)
    y0, ps0, pss0 = _layer1b(gathered, w, part.reshape(B, N, C),
                             WaE16, WaO16, b0.reshape(1, C))

    n = jnp.float32(B * N)
    mu0 = ps0 / n
    var0 = pss0 / n - mu0 * mu0
    sc0 = gamma0.reshape(1, C) / jnp.sqrt(var0 + EPS_BN)
    sh0 = beta0.reshape(1, C) - mu0 * sc0

    y1, ps1, pss1 = _layer2(y0, sc0, sh0,
                            W1.T.astype(jnp.bfloat16), b1.reshape(1, C))
    mu1 = ps1 / n
    var1 = pss1 / n - mu1 * mu1
    sc1 = gamma1.reshape(1, C) / jnp.sqrt(var1 + EPS_BN)
    sh1 = beta1.reshape(1, C) - mu1 * sc1

    return _finalize(y1, sc1.reshape(C, 1), sh1.reshape(C, 1))


# speedup vs baseline: 2.1768x; 1.0126x over previous
import functools

import jax
import jax.numpy as jnp
from jax.experimental import pallas as pl
from jax.experimental.pallas import tpu as pltpu
from jax.experimental.pallas import tpu_sc as plsc

B, N, M = 4, 4096, 1024
C = 256
IN_C = 2 * C
EPS_BN = 1e-5

TN_NN = 512
TN_MM = 512
NW = 32
GW = 128


def _nn_body(src_ref, dstT_ref, fd_ref, idx_ref, w_ref, tab_ref):
    b = pl.program_id(0)
    s = src_ref[0]
    t = dstT_ref[0]

    x = fd_ref[0].T
    v = jax.lax.bitcast_convert_type(x, jnp.int32)
    r = ((v >> 16) & 1) + jnp.int32(0x7FFF)
    b16 = ((v + r) >> 16) & jnp.int32(0xFFFF)
    half = C // 2
    tab_ref[...] = b16[:, :half] | (b16[:, half:] << 16)
    dx = s[:, 0:1] - t[0:1, :]
    dy = s[:, 1:2] - t[1:2, :]
    dz = s[:, 2:3] - t[2:3, :]
    d2 = dx * dx + dy * dy + dz * dz
    inf = jnp.float32(jnp.inf)
    v1 = jnp.min(d2, axis=1, keepdims=True)
    v2 = jnp.min(jnp.where(d2 > v1, d2, inf), axis=1, keepdims=True)
    v3 = jnp.min(jnp.where(d2 > v2, d2, inf), axis=1, keepdims=True)
    iota = jax.lax.broadcasted_iota(jnp.int32, d2.shape, 1).astype(jnp.float32)
    i1 = jnp.sum(jnp.where(d2 == v1, iota, 0.0), axis=1, keepdims=True)
    i2 = jnp.sum(jnp.where(d2 == v2, iota, 0.0), axis=1, keepdims=True)
    i3 = jnp.sum(jnp.where(d2 == v3, iota, 0.0), axis=1, keepdims=True)
    ii = jnp.concatenate([i1, i2, i3], axis=1).astype(jnp.int32)
    ii = jnp.minimum(ii, M - 1)
    vv = jnp.concatenate([v1, v2, v3], axis=1)
    d3 = jnp.sqrt(vv) + 1e-8
    w = 1.0 / d3
    w = w / jnp.sum(w, axis=1, keepdims=True)
    idx_ref[0] = ii.T + b * M
    w_ref[0] = w


def _three_nn(xyz_src, xyz_dstT, feat_dst):
    steps = N // TN_NN
    TM = M // steps
    return pl.pallas_call(
        _nn_body,
        grid=(B, steps),
        in_specs=[
            pl.BlockSpec((1, TN_NN, 3), lambda b, i: (b, i, 0)),
            pl.BlockSpec((1, 3, M), lambda b, i: (b, 0, 0)),
            pl.BlockSpec((1, C, TM), lambda b, i: (b, 0, i)),
        ],
        out_specs=[
            pl.BlockSpec((1, 3, TN_NN), lambda b, i: (b, 0, i)),
            pl.BlockSpec((1, TN_NN, 3), lambda b, i: (b, i, 0)),
            pl.BlockSpec((TM, C // 2), lambda b, i, s=steps: (b * s + i, 0)),
        ],
        out_shape=[
            jax.ShapeDtypeStruct((B, 3, N), jnp.int32),
            jax.ShapeDtypeStruct((B, N, 3), jnp.float32),
            jax.ShapeDtypeStruct((B * M, C // 2), jnp.int32),
        ],
    )(xyz_src, xyz_dstT, feat_dst)


def _sc_gather(table, gidx):
    NI = gidx.shape[0]
    per_w = NI // NW
    nch = per_w // GW
    mesh = plsc.VectorSubcoreMesh(core_axis_name="c", subcore_axis_name="s")

    @functools.partial(
        pl.kernel,
        mesh=mesh,
        out_type=jax.ShapeDtypeStruct((NI, C // 2), jnp.int32),
        scratch_types=[
            pltpu.VMEM((per_w,), jnp.int32),
            pltpu.VMEM((GW, C // 2), jnp.int32),
            pltpu.VMEM((GW, C // 2), jnp.int32),
            pltpu.SemaphoreType.DMA,
            pltpu.SemaphoreType.DMA,
            pltpu.SemaphoreType.DMA,
            pltpu.SemaphoreType.DMA,
        ],
    )
    def k(table_hbm, idx_hbm, out_hbm, idx_v, rows0, rows1,
          gsem0, gsem1, wsem0, wsem1):
        wid = jax.lax.axis_index("s") * 2 + jax.lax.axis_index("c")
        base = wid * per_w
        pltpu.sync_copy(idx_hbm.at[pl.ds(base, per_w)], idx_v)
        rows = (rows0, rows1)
        gsem = (gsem0, gsem1)
        wsem = (wsem0, wsem1)
        gcp = [None, None]
        wcp = [None, None]
        for c in range(nch):
            s = c % 2
            if c >= 2:
                wcp[s].wait()
            gcp[s] = pltpu.async_copy(
                table_hbm.at[idx_v.at[pl.ds(c * GW, GW)]], rows[s], gsem[s])
            if c >= 1:
                t = (c - 1) % 2
                gcp[t].wait()
                wcp[t] = pltpu.async_copy(
                    rows[t], out_hbm.at[pl.ds(base + (c - 1) * GW, GW)],
                    wsem[t])
        s = (nch - 1) % 2
        gcp[s].wait()
        pltpu.sync_copy(rows[s], out_hbm.at[pl.ds(base + (nch - 1) * GW, GW)])
        wcp[(nch - 2) % 2].wait()

    return k(table, gidx)


def _l1a_body(fsrc_ref, wbt_ref, p_ref):
    ft = fsrc_ref[0].T
    p_ref[0] = jnp.dot(ft.astype(jnp.bfloat16), wbt_ref[...],
                       preferred_element_type=jnp.float32)


def _layer1a(feat_src, W0bT16):
    steps = N // TN_MM
    return pl.pallas_call(
        _l1a_body,
        grid=(B, steps),
        in_specs=[
            pl.BlockSpec((1, C, TN_MM), lambda b, i: (b, 0, i)),
            pl.BlockSpec((C, C), lambda b, i: (0, 0)),
        ],
        out_specs=pl.BlockSpec((1, TN_MM, C), lambda b, i: (b, i, 0)),
        out_shape=jax.ShapeDtypeStruct((B, N, C), jnp.float32),
    )(feat_src, W0bT16)


def _l1b_body(g_ref, w_ref, part_ref, wae_ref, wao_ref, b0_ref,
              y_ref, ps_ref, pss_ref):
    w = w_ref[0]
    g = g_ref[0]
    ge = jax.lax.bitcast_convert_type(g << 16, jnp.float32)
    go = jax.lax.bitcast_convert_type(g & jnp.int32(-65536), jnp.float32)
    ie = (ge[0] * w[:, 0:1] + ge[1] * w[:, 1:2] + ge[2] * w[:, 2:3])
    io = (go[0] * w[:, 0:1] + go[1] * w[:, 1:2] + go[2] * w[:, 2:3])
    y = jnp.dot(ie.astype(jnp.bfloat16), wae_ref[...],
                preferred_element_type=jnp.float32)
    y = y + jnp.dot(io.astype(jnp.bfloat16), wao_ref[...],
                    preferred_element_type=jnp.float32)
    y = y + part_ref[0] + b0_ref[...]
    y_ref[0] = y

    @pl.when((pl.program_id(0) == 0) & (pl.program_id(1) == 0))
    def _():
        ps_ref[...] = jnp.zeros_like(ps_ref)
        pss_ref[...] = jnp.zeros_like(pss_ref)

    ps_ref[...] += jnp.sum(y, axis=0, keepdims=True)
    pss_ref[...] += jnp.sum(y * y, axis=0, keepdims=True)


def _layer1b(gathered, w, part, WaE16, WaO16, b0row):
    steps = N // TN_MM
    return pl.pallas_call(
        _l1b_body,
        grid=(B, steps),
        in_specs=[
            pl.BlockSpec((1, 3, TN_MM, C // 2), lambda b, i: (b, 0, i, 0)),
            pl.BlockSpec((1, TN_MM, 3), lambda b, i: (b, i, 0)),
            pl.BlockSpec((1, TN_MM, C), lambda b, i: (b, i, 0)),
            pl.BlockSpec((C // 2, C), lambda b, i: (0, 0)),
            pl.BlockSpec((C // 2, C), lambda b, i: (0, 0)),
            pl.BlockSpec((1, C), lambda b, i: (0, 0)),
        ],
        out_specs=[
            pl.BlockSpec((1, TN_MM, C), lambda b, i: (b, i, 0)),
            pl.BlockSpec((1, C), lambda b, i: (0, 0)),
            pl.BlockSpec((1, C), lambda b, i: (0, 0)),
        ],
        out_shape=[
            jax.ShapeDtypeStruct((B, N, C), jnp.float32),
            jax.ShapeDtypeStruct((1, C), jnp.float32),
            jax.ShapeDtypeStruct((1, C), jnp.float32),
        ],
    )(gathered, w, part, WaE16, WaO16, b0row)


def _l2_body(y0_ref, sc_ref, sh_ref, w1t_ref, b1_ref, y_ref, ps_ref, pss_ref):
    h = jnp.maximum(y0_ref[0] * sc_ref[...] + sh_ref[...], 0.0)
    y = jnp.dot(h.astype(jnp.bfloat16), w1t_ref[...],
                preferred_element_type=jnp.float32)
    y = y + b1_ref[...]
    y_ref[0] = y.T

    @pl.when((pl.program_id(0) == 0) & (pl.program_id(1) == 0))
    def _():
        ps_ref[...] = jnp.zeros_like(ps_ref)
        pss_ref[...] = jnp.zeros_like(pss_ref)

    ps_ref[...] += jnp.sum(y, axis=0, keepdims=True)
    pss_ref[...] += jnp.sum(y * y, axis=0, keepdims=True)


def _layer2(y0, sc0, sh0, W1T16, b1row):
    steps = N // TN_MM
    return pl.pallas_call(
        _l2_body,
        grid=(B, steps),
        in_specs=[
            pl.BlockSpec((1, TN_MM, C), lambda b, i: (b, i, 0)),
            pl.BlockSpec((1, C), lambda b, i: (0, 0)),
            pl.BlockSpec((1, C), lambda b, i: (0, 0)),
            pl.BlockSpec((C, C), lambda b, i: (0, 0)),
            pl.BlockSpec((1, C), lambda b, i: (0, 0)),
        ],
        out_specs=[
            pl.BlockSpec((1, C, TN_MM), lambda b, i: (b, 0, i)),
            pl.BlockSpec((1, C), lambda b, i: (0, 0)),
            pl.BlockSpec((1, C), lambda b, i: (0, 0)),
        ],
        out_shape=[
            jax.ShapeDtypeStruct((B, C, N), jnp.float32),
            jax.ShapeDtypeStruct((1, C), jnp.float32),
            jax.ShapeDtypeStruct((1, C), jnp.float32),
        ],
    )(y0, sc0, sh0, W1T16, b1row)


def _out_body(y1_ref, sc_ref, sh_ref, o_ref):
    o_ref[0] = jnp.maximum(y1_ref[0] * sc_ref[...] + sh_ref[...], 0.0)


def _finalize(y1, sc1, sh1):
    TF = 2048
    return pl.pallas_call(
        _out_body,
        grid=(B, N // TF),
        in_specs=[
            pl.BlockSpec((1, C, TF), lambda b, i: (b, 0, i)),
            pl.BlockSpec((C, 1), lambda b, i: (0, 0)),
            pl.BlockSpec((C, 1), lambda b, i: (0, 0)),
        ],
        out_specs=pl.BlockSpec((1, C, TF), lambda b, i: (b, 0, i)),
        out_shape=jax.ShapeDtypeStruct((B, C, N), jnp.float32),
    )(y1, sc1, sh1)


def kernel(xyz_src, xyz_dst, feat_src, feat_dst,
           W0, b0, gamma0, beta0, W1, b1, gamma1, beta1):
    xyz_dstT = jnp.transpose(xyz_dst, (0, 2, 1))
    W0T = W0.T
    WaE16 = W0T[:C // 2].astype(jnp.bfloat16)
    WaO16 = W0T[C // 2:C].astype(jnp.bfloat16)
    W0bT16 = W0T[C:].astype(jnp.bfloat16)

    idxT, w, table32 = _three_nn(xyz_src, xyz_dstT, feat_dst)
    g32 = _sc_gather(table32, idxT.reshape(B * 3 * N))
    gathered = g32.reshape(B, 3, N, C // 2)

    part = _layer1a(feat_src, W0bT16)
    y0, ps0, pss0 = _layer1b(gathered, w, part,
                             WaE16, WaO16, b0.reshape(1, C))

    n = jnp.float32(B * N)
    mu0 = ps0 / n
    var0 = pss0 / n - mu0 * mu0
    sc0 = gamma0.reshape(1, C) / jnp.sqrt(var0 + EPS_BN)
    sh0 = beta0.reshape(1, C) - mu0 * sc0

    y1, ps1, pss1 = _layer2(y0, sc0, sh0,
                            W1.T.astype(jnp.bfloat16), b1.reshape(1, C))
    mu1 = ps1 / n
    var1 = pss1 / n - mu1 * mu1
    sc1 = gamma1.reshape(1, C) / jnp.sqrt(var1 + EPS_BN)
    sh1 = beta1.reshape(1, C) - mu1 * sc1

    return _finalize(y1, sc1.reshape(C, 1), sh1.reshape(C, 1))


# speedup vs baseline: 2.4736x; 1.1363x over previous
import functools

import jax
import jax.numpy as jnp
from jax.experimental import pallas as pl
from jax.experimental.pallas import tpu as pltpu
from jax.experimental.pallas import tpu_sc as plsc

B, N, M = 4, 4096, 1024
C = 256
IN_C = 2 * C
EPS_BN = 1e-5

TN_NN = 512
TN_MM = 1024
NW = 32
GW = 128


def _nn_body(src_ref, dstT_ref, fd_ref, idx_ref, w_ref, tab_ref):
    b = pl.program_id(0)
    s = src_ref[0]
    t = dstT_ref[0]

    x = fd_ref[0].T
    v = jax.lax.bitcast_convert_type(x, jnp.int32)
    r = ((v >> 16) & 1) + jnp.int32(0x7FFF)
    b16 = ((v + r) >> 16) & jnp.int32(0xFFFF)
    half = C // 2
    tab_ref[...] = b16[:, :half] | (b16[:, half:] << 16)
    dx = s[:, 0:1] - t[0:1, :]
    dy = s[:, 1:2] - t[1:2, :]
    dz = s[:, 2:3] - t[2:3, :]
    d2 = dx * dx + dy * dy + dz * dz
    inf = jnp.float32(jnp.inf)
    v1 = jnp.min(d2, axis=1, keepdims=True)
    v2 = jnp.min(jnp.where(d2 > v1, d2, inf), axis=1, keepdims=True)
    v3 = jnp.min(jnp.where(d2 > v2, d2, inf), axis=1, keepdims=True)
    iota = jax.lax.broadcasted_iota(jnp.int32, d2.shape, 1).astype(jnp.float32)
    i1 = jnp.sum(jnp.where(d2 == v1, iota, 0.0), axis=1, keepdims=True)
    i2 = jnp.sum(jnp.where(d2 == v2, iota, 0.0), axis=1, keepdims=True)
    i3 = jnp.sum(jnp.where(d2 == v3, iota, 0.0), axis=1, keepdims=True)
    ii = jnp.concatenate([i1, i2, i3], axis=1).astype(jnp.int32)
    ii = jnp.minimum(ii, M - 1)
    vv = jnp.concatenate([v1, v2, v3], axis=1)
    d3 = jnp.sqrt(vv) + 1e-8
    w = 1.0 / d3
    w = w / jnp.sum(w, axis=1, keepdims=True)
    idx_ref[0] = ii.T + b * M
    w_ref[0] = w


def _three_nn(xyz_src, xyz_dstT, feat_dst):
    steps = N // TN_NN
    TM = M // steps
    return pl.pallas_call(
        _nn_body,
        grid=(B, steps),
        in_specs=[
            pl.BlockSpec((1, TN_NN, 3), lambda b, i: (b, i, 0)),
            pl.BlockSpec((1, 3, M), lambda b, i: (b, 0, 0)),
            pl.BlockSpec((1, C, TM), lambda b, i: (b, 0, i)),
        ],
        out_specs=[
            pl.BlockSpec((1, 3, TN_NN), lambda b, i: (b, 0, i)),
            pl.BlockSpec((1, TN_NN, 3), lambda b, i: (b, i, 0)),
            pl.BlockSpec((TM, C // 2), lambda b, i, s=steps: (b * s + i, 0)),
        ],
        out_shape=[
            jax.ShapeDtypeStruct((B, 3, N), jnp.int32),
            jax.ShapeDtypeStruct((B, N, 3), jnp.float32),
            jax.ShapeDtypeStruct((B * M, C // 2), jnp.int32),
        ],
    )(xyz_src, xyz_dstT, feat_dst)


def _sc_gather(table, gidx):
    NI = gidx.shape[0]
    per_w = NI // NW
    nch = per_w // GW
    mesh = plsc.VectorSubcoreMesh(core_axis_name="c", subcore_axis_name="s")

    @functools.partial(
        pl.kernel,
        mesh=mesh,
        out_type=jax.ShapeDtypeStruct((NI, C // 2), jnp.int32),
        scratch_types=[
            pltpu.VMEM((per_w,), jnp.int32),
            pltpu.VMEM((GW, C // 2), jnp.int32),
            pltpu.VMEM((GW, C // 2), jnp.int32),
            pltpu.SemaphoreType.DMA,
            pltpu.SemaphoreType.DMA,
            pltpu.SemaphoreType.DMA,
            pltpu.SemaphoreType.DMA,
        ],
    )
    def k(table_hbm, idx_hbm, out_hbm, idx_v, rows0, rows1,
          gsem0, gsem1, wsem0, wsem1):
        wid = jax.lax.axis_index("s") * 2 + jax.lax.axis_index("c")
        base = wid * per_w
        pltpu.sync_copy(idx_hbm.at[pl.ds(base, per_w)], idx_v)
        rows = (rows0, rows1)
        gsem = (gsem0, gsem1)
        wsem = (wsem0, wsem1)
        gcp = [None, None]
        wcp = [None, None]
        for c in range(nch):
            s = c % 2
            if c >= 2:
                wcp[s].wait()
            gcp[s] = pltpu.async_copy(
                table_hbm.at[idx_v.at[pl.ds(c * GW, GW)]], rows[s], gsem[s])
            if c >= 1:
                t = (c - 1) % 2
                gcp[t].wait()
                wcp[t] = pltpu.async_copy(
                    rows[t], out_hbm.at[pl.ds(base + (c - 1) * GW, GW)],
                    wsem[t])
        s = (nch - 1) % 2
        gcp[s].wait()
        pltpu.sync_copy(rows[s], out_hbm.at[pl.ds(base + (nch - 1) * GW, GW)])
        wcp[(nch - 2) % 2].wait()

    return k(table, gidx)


def _l1a_body(fsrc_ref, wbt_ref, p_ref):
    ft = fsrc_ref[0].T
    p_ref[0] = jnp.dot(ft.astype(jnp.bfloat16), wbt_ref[...],
                       preferred_element_type=jnp.float32)


def _layer1a(feat_src, W0bT16):
    steps = N // TN_MM
    return pl.pallas_call(
        _l1a_body,
        grid=(B, steps),
        in_specs=[
            pl.BlockSpec((1, C, TN_MM), lambda b, i: (b, 0, i)),
            pl.BlockSpec((C, C), lambda b, i: (0, 0)),
        ],
        out_specs=pl.BlockSpec((1, TN_MM, C), lambda b, i: (b, i, 0)),
        out_shape=jax.ShapeDtypeStruct((B, N, C), jnp.float32),
    )(feat_src, W0bT16)


def _l1b_body(g_ref, w_ref, part_ref, wae_ref, wao_ref, b0_ref,
              y_ref, ps_ref, pss_ref):
    w = w_ref[0]
    g = g_ref[0]
    ge = jax.lax.bitcast_convert_type(g << 16, jnp.float32)
    go = jax.lax.bitcast_convert_type(g & jnp.int32(-65536), jnp.float32)
    ie = (ge[0] * w[:, 0:1] + ge[1] * w[:, 1:2] + ge[2] * w[:, 2:3])
    io = (go[0] * w[:, 0:1] + go[1] * w[:, 1:2] + go[2] * w[:, 2:3])
    y = jnp.dot(ie.astype(jnp.bfloat16), wae_ref[...],
                preferred_element_type=jnp.float32)
    y = y + jnp.dot(io.astype(jnp.bfloat16), wao_ref[...],
                    preferred_element_type=jnp.float32)
    y = y + part_ref[0] + b0_ref[...]
    y_ref[0] = y

    @pl.when((pl.program_id(0) == 0) & (pl.program_id(1) == 0))
    def _():
        ps_ref[...] = jnp.zeros_like(ps_ref)
        pss_ref[...] = jnp.zeros_like(pss_ref)

    ps_ref[...] += jnp.sum(y, axis=0, keepdims=True)
    pss_ref[...] += jnp.sum(y * y, axis=0, keepdims=True)


def _layer1b(gathered, w, part, WaE16, WaO16, b0row):
    steps = N // TN_MM
    return pl.pallas_call(
        _l1b_body,
        grid=(B, steps),
        in_specs=[
            pl.BlockSpec((1, 3, TN_MM, C // 2), lambda b, i: (b, 0, i, 0)),
            pl.BlockSpec((1, TN_MM, 3), lambda b, i: (b, i, 0)),
            pl.BlockSpec((1, TN_MM, C), lambda b, i: (b, i, 0)),
            pl.BlockSpec((C // 2, C), lambda b, i: (0, 0)),
            pl.BlockSpec((C // 2, C), lambda b, i: (0, 0)),
            pl.BlockSpec((1, C), lambda b, i: (0, 0)),
        ],
        out_specs=[
            pl.BlockSpec((1, TN_MM, C), lambda b, i: (b, i, 0)),
            pl.BlockSpec((1, C), lambda b, i: (0, 0)),
            pl.BlockSpec((1, C), lambda b, i: (0, 0)),
        ],
        out_shape=[
            jax.ShapeDtypeStruct((B, N, C), jnp.float32),
            jax.ShapeDtypeStruct((1, C), jnp.float32),
            jax.ShapeDtypeStruct((1, C), jnp.float32),
        ],
    )(gathered, w, part, WaE16, WaO16, b0row)


def _l2_body(y0_ref, sc_ref, sh_ref, w1t_ref, b1_ref, y_ref, ps_ref, pss_ref):
    h = jnp.maximum(y0_ref[0] * sc_ref[...] + sh_ref[...], 0.0)
    y = jnp.dot(h.astype(jnp.bfloat16), w1t_ref[...],
                preferred_element_type=jnp.float32)
    y = y + b1_ref[...]
    y_ref[0] = y.T

    @pl.when((pl.program_id(0) == 0) & (pl.program_id(1) == 0))
    def _():
        ps_ref[...] = jnp.zeros_like(ps_ref)
        pss_ref[...] = jnp.zeros_like(pss_ref)

    ps_ref[...] += jnp.sum(y, axis=0, keepdims=True)
    pss_ref[...] += jnp.sum(y * y, axis=0, keepdims=True)


def _layer2(y0, sc0, sh0, W1T16, b1row):
    steps = N // TN_MM
    return pl.pallas_call(
        _l2_body,
        grid=(B, steps),
        in_specs=[
            pl.BlockSpec((1, TN_MM, C), lambda b, i: (b, i, 0)),
            pl.BlockSpec((1, C), lambda b, i: (0, 0)),
            pl.BlockSpec((1, C), lambda b, i: (0, 0)),
            pl.BlockSpec((C, C), lambda b, i: (0, 0)),
            pl.BlockSpec((1, C), lambda b, i: (0, 0)),
        ],
        out_specs=[
            pl.BlockSpec((1, C, TN_MM), lambda b, i: (b, 0, i)),
            pl.BlockSpec((1, C), lambda b, i: (0, 0)),
            pl.BlockSpec((1, C), lambda b, i: (0, 0)),
        ],
        out_shape=[
            jax.ShapeDtypeStruct((B, C, N), jnp.float32),
            jax.ShapeDtypeStruct((1, C), jnp.float32),
            jax.ShapeDtypeStruct((1, C), jnp.float32),
        ],
    )(y0, sc0, sh0, W1T16, b1row)


def _out_body(y1_ref, sc_ref, sh_ref, o_ref):
    o_ref[0] = jnp.maximum(y1_ref[0] * sc_ref[...] + sh_ref[...], 0.0)


def _finalize(y1, sc1, sh1):
    TF = 2048
    return pl.pallas_call(
        _out_body,
        grid=(B, N // TF),
        in_specs=[
            pl.BlockSpec((1, C, TF), lambda b, i: (b, 0, i)),
            pl.BlockSpec((C, 1), lambda b, i: (0, 0)),
            pl.BlockSpec((C, 1), lambda b, i: (0, 0)),
        ],
        out_specs=pl.BlockSpec((1, C, TF), lambda b, i: (b, 0, i)),
        out_shape=jax.ShapeDtypeStruct((B, C, N), jnp.float32),
    )(y1, sc1, sh1)


def kernel(xyz_src, xyz_dst, feat_src, feat_dst,
           W0, b0, gamma0, beta0, W1, b1, gamma1, beta1):
    xyz_dstT = jnp.transpose(xyz_dst, (0, 2, 1))
    W0T = W0.T
    WaE16 = W0T[:C // 2].astype(jnp.bfloat16)
    WaO16 = W0T[C // 2:C].astype(jnp.bfloat16)
    W0bT16 = W0T[C:].astype(jnp.bfloat16)

    idxT, w, table32 = _three_nn(xyz_src, xyz_dstT, feat_dst)
    g32 = _sc_gather(table32, idxT.reshape(B * 3 * N))
    gathered = g32.reshape(B, 3, N, C // 2)

    part = _layer1a(feat_src, W0bT16)
    y0, ps0, pss0 = _layer1b(gathered, w, part,
                             WaE16, WaO16, b0.reshape(1, C))

    n = jnp.float32(B * N)
    mu0 = ps0 / n
    var0 = pss0 / n - mu0 * mu0
    sc0 = gamma0.reshape(1, C) / jnp.sqrt(var0 + EPS_BN)
    sh0 = beta0.reshape(1, C) - mu0 * sc0

    y1, ps1, pss1 = _layer2(y0, sc0, sh0,
                            W1.T.astype(jnp.bfloat16), b1.reshape(1, C))
    mu1 = ps1 / n
    var1 = pss1 / n - mu1 * mu1
    sc1 = gamma1.reshape(1, C) / jnp.sqrt(var1 + EPS_BN)
    sh1 = beta1.reshape(1, C) - mu1 * sc1

    return _finalize(y1, sc1.reshape(C, 1), sh1.reshape(C, 1))


# speedup vs baseline: 2.4981x; 1.0099x over previous
import functools

import jax
import jax.numpy as jnp
from jax.experimental import pallas as pl
from jax.experimental.pallas import tpu as pltpu
from jax.experimental.pallas import tpu_sc as plsc

B, N, M = 4, 4096, 1024
C = 256
IN_C = 2 * C
EPS_BN = 1e-5

TN_NN = 1024
TN_MM = 1024
NW = 32
GW = 128


def _nn_body(src_ref, dstT_ref, fd_ref, idx_ref, w_ref, tab_ref):
    b = pl.program_id(0)
    s = src_ref[0]
    t = dstT_ref[0]

    x = fd_ref[0].T
    v = jax.lax.bitcast_convert_type(x, jnp.int32)
    r = ((v >> 16) & 1) + jnp.int32(0x7FFF)
    b16 = ((v + r) >> 16) & jnp.int32(0xFFFF)
    half = C // 2
    tab_ref[...] = b16[:, :half] | (b16[:, half:] << 16)
    dx = s[:, 0:1] - t[0:1, :]
    dy = s[:, 1:2] - t[1:2, :]
    dz = s[:, 2:3] - t[2:3, :]
    d2 = dx * dx + dy * dy + dz * dz
    inf = jnp.float32(jnp.inf)
    v1 = jnp.min(d2, axis=1, keepdims=True)
    v2 = jnp.min(jnp.where(d2 > v1, d2, inf), axis=1, keepdims=True)
    v3 = jnp.min(jnp.where(d2 > v2, d2, inf), axis=1, keepdims=True)
    iota = jax.lax.broadcasted_iota(jnp.int32, d2.shape, 1).astype(jnp.float32)
    i1 = jnp.sum(jnp.where(d2 == v1, iota, 0.0), axis=1, keepdims=True)
    i2 = jnp.sum(jnp.where(d2 == v2, iota, 0.0), axis=1, keepdims=True)
    i3 = jnp.sum(jnp.where(d2 == v3, iota, 0.0), axis=1, keepdims=True)
    ii = jnp.concatenate([i1, i2, i3], axis=1).astype(jnp.int32)
    ii = jnp.minimum(ii, M - 1)
    vv = jnp.concatenate([v1, v2, v3], axis=1)
    d3 = jnp.sqrt(vv) + 1e-8
    w = 1.0 / d3
    w = w / jnp.sum(w, axis=1, keepdims=True)
    idx_ref[0] = ii.T + b * M
    w_ref[0] = w


def _three_nn(xyz_src, xyz_dstT, feat_dst):
    steps = N // TN_NN
    TM = M // steps
    return pl.pallas_call(
        _nn_body,
        grid=(B, steps),
        in_specs=[
            pl.BlockSpec((1, TN_NN, 3), lambda b, i: (b, i, 0)),
            pl.BlockSpec((1, 3, M), lambda b, i: (b, 0, 0)),
            pl.BlockSpec((1, C, TM), lambda b, i: (b, 0, i)),
        ],
        out_specs=[
            pl.BlockSpec((1, 3, TN_NN), lambda b, i: (b, 0, i)),
            pl.BlockSpec((1, TN_NN, 3), lambda b, i: (b, i, 0)),
            pl.BlockSpec((TM, C // 2), lambda b, i, s=steps: (b * s + i, 0)),
        ],
        out_shape=[
            jax.ShapeDtypeStruct((B, 3, N), jnp.int32),
            jax.ShapeDtypeStruct((B, N, 3), jnp.float32),
            jax.ShapeDtypeStruct((B * M, C // 2), jnp.int32),
        ],
    )(xyz_src, xyz_dstT, feat_dst)


def _sc_gather(table, gidx):
    NI = gidx.shape[0]
    per_w = NI // NW
    nch = per_w // GW
    mesh = plsc.VectorSubcoreMesh(core_axis_name="c", subcore_axis_name="s")

    @functools.partial(
        pl.kernel,
        mesh=mesh,
        out_type=jax.ShapeDtypeStruct((NI, C // 2), jnp.int32),
        scratch_types=[
            pltpu.VMEM((per_w,), jnp.int32),
            pltpu.VMEM((GW, C // 2), jnp.int32),
            pltpu.VMEM((GW, C // 2), jnp.int32),
            pltpu.SemaphoreType.DMA,
            pltpu.SemaphoreType.DMA,
            pltpu.SemaphoreType.DMA,
            pltpu.SemaphoreType.DMA,
        ],
    )
    def k(table_hbm, idx_hbm, out_hbm, idx_v, rows0, rows1,
          gsem0, gsem1, wsem0, wsem1):
        wid = jax.lax.axis_index("s") * 2 + jax.lax.axis_index("c")
        base = wid * per_w
        pltpu.sync_copy(idx_hbm.at[pl.ds(base, per_w)], idx_v)
        rows = (rows0, rows1)
        gsem = (gsem0, gsem1)
        wsem = (wsem0, wsem1)
        gcp = [None, None]
        wcp = [None, None]
        for c in range(nch):
            s = c % 2
            if c >= 2:
                wcp[s].wait()
            gcp[s] = pltpu.async_copy(
                table_hbm.at[idx_v.at[pl.ds(c * GW, GW)]], rows[s], gsem[s])
            if c >= 1:
                t = (c - 1) % 2
                gcp[t].wait()
                wcp[t] = pltpu.async_copy(
                    rows[t], out_hbm.at[pl.ds(base + (c - 1) * GW, GW)],
                    wsem[t])
        s = (nch - 1) % 2
        gcp[s].wait()
        pltpu.sync_copy(rows[s], out_hbm.at[pl.ds(base + (nch - 1) * GW, GW)])
        wcp[(nch - 2) % 2].wait()

    return k(table, gidx)


def _l1a_body(fsrc_ref, wbt_ref, p_ref):
    ft = fsrc_ref[0].T
    p_ref[0] = jnp.dot(ft.astype(jnp.bfloat16), wbt_ref[...],
                       preferred_element_type=jnp.float32)


def _layer1a(feat_src, W0bT16):
    steps = N // TN_MM
    return pl.pallas_call(
        _l1a_body,
        grid=(B, steps),
        in_specs=[
            pl.BlockSpec((1, C, TN_MM), lambda b, i: (b, 0, i)),
            pl.BlockSpec((C, C), lambda b, i: (0, 0)),
        ],
        out_specs=pl.BlockSpec((1, TN_MM, C), lambda b, i: (b, i, 0)),
        out_shape=jax.ShapeDtypeStruct((B, N, C), jnp.float32),
    )(feat_src, W0bT16)


def _l1b_body(g_ref, w_ref, part_ref, wae_ref, wao_ref, b0_ref,
              y_ref, ps_ref, pss_ref):
    w = w_ref[0]
    g = g_ref[0]
    ge = jax.lax.bitcast_convert_type(g << 16, jnp.float32)
    go = jax.lax.bitcast_convert_type(g & jnp.int32(-65536), jnp.float32)
    ie = (ge[0] * w[:, 0:1] + ge[1] * w[:, 1:2] + ge[2] * w[:, 2:3])
    io = (go[0] * w[:, 0:1] + go[1] * w[:, 1:2] + go[2] * w[:, 2:3])
    y = jnp.dot(ie.astype(jnp.bfloat16), wae_ref[...],
                preferred_element_type=jnp.float32)
    y = y + jnp.dot(io.astype(jnp.bfloat16), wao_ref[...],
                    preferred_element_type=jnp.float32)
    y = y + part_ref[0] + b0_ref[...]
    y_ref[0] = y

    @pl.when((pl.program_id(0) == 0) & (pl.program_id(1) == 0))
    def _():
        ps_ref[...] = jnp.zeros_like(ps_ref)
        pss_ref[...] = jnp.zeros_like(pss_ref)

    ps_ref[...] += jnp.sum(y, axis=0, keepdims=True)
    pss_ref[...] += jnp.sum(y * y, axis=0, keepdims=True)


def _layer1b(gathered, w, part, WaE16, WaO16, b0row):
    steps = N // TN_MM
    return pl.pallas_call(
        _l1b_body,
        grid=(B, steps),
        in_specs=[
            pl.BlockSpec((1, 3, TN_MM, C // 2), lambda b, i: (b, 0, i, 0)),
            pl.BlockSpec((1, TN_MM, 3), lambda b, i: (b, i, 0)),
            pl.BlockSpec((1, TN_MM, C), lambda b, i: (b, i, 0)),
            pl.BlockSpec((C // 2, C), lambda b, i: (0, 0)),
            pl.BlockSpec((C // 2, C), lambda b, i: (0, 0)),
            pl.BlockSpec((1, C), lambda b, i: (0, 0)),
        ],
        out_specs=[
            pl.BlockSpec((1, TN_MM, C), lambda b, i: (b, i, 0)),
            pl.BlockSpec((1, C), lambda b, i: (0, 0)),
            pl.BlockSpec((1, C), lambda b, i: (0, 0)),
        ],
        out_shape=[
            jax.ShapeDtypeStruct((B, N, C), jnp.float32),
            jax.ShapeDtypeStruct((1, C), jnp.float32),
            jax.ShapeDtypeStruct((1, C), jnp.float32),
        ],
    )(gathered, w, part, WaE16, WaO16, b0row)


def _l2_body(y0_ref, sc_ref, sh_ref, w1t_ref, b1_ref, y_ref, ps_ref, pss_ref):
    h = jnp.maximum(y0_ref[0] * sc_ref[...] + sh_ref[...], 0.0)
    y = jnp.dot(h.astype(jnp.bfloat16), w1t_ref[...],
                preferred_element_type=jnp.float32)
    y = y + b1_ref[...]
    y_ref[0] = y.T

    @pl.when((pl.program_id(0) == 0) & (pl.program_id(1) == 0))
    def _():
        ps_ref[...] = jnp.zeros_like(ps_ref)
        pss_ref[...] = jnp.zeros_like(pss_ref)

    ps_ref[...] += jnp.sum(y, axis=0, keepdims=True)
    pss_ref[...] += jnp.sum(y * y, axis=0, keepdims=True)


def _layer2(y0, sc0, sh0, W1T16, b1row):
    steps = N // TN_MM
    return pl.pallas_call(
        _l2_body,
        grid=(B, steps),
        in_specs=[
            pl.BlockSpec((1, TN_MM, C), lambda b, i: (b, i, 0)),
            pl.BlockSpec((1, C), lambda b, i: (0, 0)),
            pl.BlockSpec((1, C), lambda b, i: (0, 0)),
            pl.BlockSpec((C, C), lambda b, i: (0, 0)),
            pl.BlockSpec((1, C), lambda b, i: (0, 0)),
        ],
        out_specs=[
            pl.BlockSpec((1, C, TN_MM), lambda b, i: (b, 0, i)),
            pl.BlockSpec((1, C), lambda b, i: (0, 0)),
            pl.BlockSpec((1, C), lambda b, i: (0, 0)),
        ],
        out_shape=[
            jax.ShapeDtypeStruct((B, C, N), jnp.float32),
            jax.ShapeDtypeStruct((1, C), jnp.float32),
            jax.ShapeDtypeStruct((1, C), jnp.float32),
        ],
    )(y0, sc0, sh0, W1T16, b1row)


def _out_body(y1_ref, sc_ref, sh_ref, o_ref):
    o_ref[0] = jnp.maximum(y1_ref[0] * sc_ref[...] + sh_ref[...], 0.0)


def _finalize(y1, sc1, sh1):
    TF = 2048
    return pl.pallas_call(
        _out_body,
        grid=(B, N // TF),
        in_specs=[
            pl.BlockSpec((1, C, TF), lambda b, i: (b, 0, i)),
            pl.BlockSpec((C, 1), lambda b, i: (0, 0)),
            pl.BlockSpec((C, 1), lambda b, i: (0, 0)),
        ],
        out_specs=pl.BlockSpec((1, C, TF), lambda b, i: (b, 0, i)),
        out_shape=jax.ShapeDtypeStruct((B, C, N), jnp.float32),
    )(y1, sc1, sh1)


def kernel(xyz_src, xyz_dst, feat_src, feat_dst,
           W0, b0, gamma0, beta0, W1, b1, gamma1, beta1):
    xyz_dstT = jnp.transpose(xyz_dst, (0, 2, 1))
    W0T = W0.T
    WaE16 = W0T[:C // 2].astype(jnp.bfloat16)
    WaO16 = W0T[C // 2:C].astype(jnp.bfloat16)
    W0bT16 = W0T[C:].astype(jnp.bfloat16)

    idxT, w, table32 = _three_nn(xyz_src, xyz_dstT, feat_dst)
    g32 = _sc_gather(table32, idxT.reshape(B * 3 * N))
    gathered = g32.reshape(B, 3, N, C // 2)

    part = _layer1a(feat_src, W0bT16)
    y0, ps0, pss0 = _layer1b(gathered, w, part,
                             WaE16, WaO16, b0.reshape(1, C))

    n = jnp.float32(B * N)
    mu0 = ps0 / n
    var0 = pss0 / n - mu0 * mu0
    sc0 = gamma0.reshape(1, C) / jnp.sqrt(var0 + EPS_BN)
    sh0 = beta0.reshape(1, C) - mu0 * sc0

    y1, ps1, pss1 = _layer2(y0, sc0, sh0,
                            W1.T.astype(jnp.bfloat16), b1.reshape(1, C))
    mu1 = ps1 / n
    var1 = pss1 / n - mu1 * mu1
    sc1 = gamma1.reshape(1, C) / jnp.sqrt(var1 + EPS_BN)
    sh1 = beta1.reshape(1, C) - mu1 * sc1

    return _finalize(y1, sc1.reshape(C, 1), sh1.reshape(C, 1))
